# Initial kernel scaffold; baseline (speedup 1.0000x reference)
import jax
import jax.numpy as jnp
from jax.experimental import pallas as pl


def kernel(input, weight):
    raise NotImplementedError("write your pallas kernel here")


# speedup vs baseline: 4.8090x; 4.8090x over previous
import functools

import jax
import jax.numpy as jnp
from jax import lax
from jax.experimental import pallas as pl
from jax.experimental.pallas import tpu as pltpu
from jax.experimental.pallas import tpu_sc as plsc

EMB_DIM = 32
NUM_CORES = 2
NUM_SUBCORES = 16
CHUNK = 1024


@functools.lru_cache(maxsize=None)
def _make_gather(B, D, C):
    NW = NUM_CORES * NUM_SUBCORES
    assert B % (NW * C) == 0
    b_per_w = B // NW
    n_steps = b_per_w // C
    mesh = plsc.VectorSubcoreMesh(core_axis_name="c", subcore_axis_name="s")

    @functools.partial(
        pl.kernel,
        out_type=jax.ShapeDtypeStruct((B, D), jnp.float32),
        mesh=mesh,
        scratch_types=[
            pltpu.VMEM((C,), jnp.int32),
            pltpu.VMEM((C, D), jnp.float32),
            pltpu.SemaphoreType.DMA,
        ],
        compiler_params=pltpu.CompilerParams(use_tc_tiling_on_sc=False),
    )
    def gather_kernel(idx_hbm, table_hbm, out_hbm, idx_v, rows_v, sem):
        wid = lax.axis_index("s") * NUM_CORES + lax.axis_index("c")
        base = wid * b_per_w

        def step(g, carry):
            off = base + g * C
            pltpu.sync_copy(idx_hbm.at[pl.ds(off, C)], idx_v)
            pltpu.async_copy(table_hbm.at[idx_v], rows_v, sem).wait()
            pltpu.sync_copy(rows_v, out_hbm.at[pl.ds(off, C)])
            return carry

        lax.fori_loop(0, n_steps, step, 0)

    return gather_kernel


@jax.jit
def kernel(input, weight):
    n, s = input.shape
    B = n * s
    idx = input.reshape(B).astype(jnp.int32)
    out = _make_gather(B, EMB_DIM, CHUNK)(idx, weight)
    return out.reshape(n, s, EMB_DIM)


# speedup vs baseline: 5.0461x; 1.0493x over previous
import functools

import jax
import jax.numpy as jnp
from jax import lax
from jax.experimental import pallas as pl
from jax.experimental.pallas import tpu as pltpu
from jax.experimental.pallas import tpu_sc as plsc

EMB_DIM = 32
NUM_CORES = 2
NUM_SUBCORES = 16
CHUNK = 512
NBUF = 4


@functools.lru_cache(maxsize=None)
def _make_gather(B, D, C, NBUF):
    NW = NUM_CORES * NUM_SUBCORES
    assert B % (NW * C) == 0
    b_per_w = B // NW
    n = b_per_w // C
    assert n >= NBUF
    n_slots = n + 1
    n_groups = (n_slots + NBUF - 1) // NBUF
    mesh = plsc.VectorSubcoreMesh(core_axis_name="c", subcore_axis_name="s")

    scratch = (
        [pltpu.VMEM((C,), jnp.int32)] * NBUF
        + [pltpu.VMEM((C, D), jnp.float32)] * NBUF
        + [pltpu.SemaphoreType.DMA] * (3 * NBUF)
    )

    @functools.partial(
        pl.kernel,
        out_type=jax.ShapeDtypeStruct((B, D), jnp.float32),
        mesh=mesh,
        scratch_types=scratch,
        compiler_params=pltpu.CompilerParams(use_tc_tiling_on_sc=False),
    )
    def gather_kernel(idx_hbm, table_hbm, out_hbm, *scr):
        idx_v = scr[0:NBUF]
        rows_v = scr[NBUF : 2 * NBUF]
        isem = scr[2 * NBUF : 3 * NBUF]
        gsem = scr[3 * NBUF : 4 * NBUF]
        ssem = scr[4 * NBUF : 5 * NBUF]

        wid = lax.axis_index("s") * NUM_CORES + lax.axis_index("c")
        base = wid * b_per_w

        def start_idx(g, p):
            pltpu.async_copy(idx_hbm.at[pl.ds(base + g * C, C)], idx_v[p], isem[p])

        def wait_idx(g, p):
            pltpu.make_async_copy(
                idx_hbm.at[pl.ds(base + g * C, C)], idx_v[p], isem[p]
            ).wait()

        def start_gather(p):
            pltpu.async_copy(table_hbm.at[idx_v[p]], rows_v[p], gsem[p])

        def wait_gather(p):
            pltpu.make_async_copy(table_hbm.at[idx_v[p]], rows_v[p], gsem[p]).wait()

        def start_store(g, p):
            pltpu.async_copy(rows_v[p], out_hbm.at[pl.ds(base + g * C, C)], ssem[p])

        def wait_store(g, p):
            pltpu.make_async_copy(
                rows_v[p], out_hbm.at[pl.ds(base + g * C, C)], ssem[p]
            ).wait()

        start_idx(0, 0)
        start_idx(1, 1)

        def group(j, carry):
            for p in range(NBUF):
                g = j * NBUF + p

                @pl.when(g < n)
                def _():
                    wait_idx(g, p)

                @pl.when(jnp.logical_and(g >= NBUF, g < n))
                def _():
                    wait_store(g - NBUF, p)

                @pl.when(g < n)
                def _():
                    start_gather(p)

                @pl.when(g + 2 < n)
                def _():
                    start_idx(g + 2, (p + 2) % NBUF)

                @pl.when(jnp.logical_and(g >= 1, g <= n))
                def _():
                    pm1 = (p - 1) % NBUF
                    wait_gather(pm1)
                    start_store(g - 1, pm1)

            return carry

        lax.fori_loop(0, n_groups, group, 0)

        for i in range(NBUF):
            g = n - NBUF + i
            wait_store(g, g % NBUF)

    return gather_kernel


@jax.jit
def kernel(input, weight):
    nrow, s = input.shape
    B = nrow * s
    idx = input.reshape(B).astype(jnp.int32)
    out = _make_gather(B, EMB_DIM, CHUNK, NBUF)(idx, weight)
    return out.reshape(nrow, s, EMB_DIM)
